# baseline (speedup 1.0000x reference)
import functools

import jax
import jax.numpy as jnp
from jax import lax
from jax.experimental import pallas as pl
from jax.experimental.pallas import tpu as pltpu
from jax.experimental.pallas import tpu_sc as plsc

D = 64
SCALE = 8.0
NB = 4096
NT = 200
B = NB * NT
NC = 2
NS = 16
NW = NC * NS
BPW = B // NW
C = 128
NCHUNK = BPW // C
NBUF = 10
PF = 5


def _sc_gather(idx3, table):
    mesh = plsc.VectorSubcoreMesh(core_axis_name="c", subcore_axis_name="s")

    scratch = [pltpu.VMEM((NCHUNK, C), jnp.int32)]
    scratch += [pltpu.VMEM((C, D), jnp.float32) for _ in range(NBUF)]
    scratch += [pltpu.SemaphoreType.DMA for _ in range(2 * NBUF + 1)]

    @functools.partial(
        pl.kernel,
        mesh=mesh,
        out_type=jax.ShapeDtypeStruct((B, 2 * D), jnp.float32),
        scratch_types=scratch,
        compiler_params=pltpu.CompilerParams(
            use_tc_tiling_on_sc=False, needs_layout_passes=False),
    )
    def kern(idx_hbm, tab_hbm, out_hbm, idx_v, *bufs_and_sems):
        gbuf = bufs_and_sems[:NBUF]
        gsem = bufs_and_sems[NBUF:2 * NBUF]
        ssem = bufs_and_sems[2 * NBUF:3 * NBUF]
        isem = bufs_and_sems[3 * NBUF]

        wid = lax.axis_index("s") * NC + lax.axis_index("c")
        pltpu.async_copy(idx_hbm.at[wid], idx_v, isem).wait()

        def gather(ci, p):
            pltpu.async_copy(tab_hbm.at[idx_v.at[ci]], gbuf[p], gsem[p])

        def store_wait(p):
            pltpu.make_async_copy(
                gbuf[p], out_hbm.at[pl.ds(0, C), pl.ds(0, D)], ssem[p]).wait()

        for p in range(PF):
            gather(p, p)

        def outer(i, _):
            cg = i * NBUF
            for k in range(NBUF):
                ci = cg + k
                p = k
                pltpu.make_async_copy(
                    tab_hbm.at[idx_v.at[ci]], gbuf[p], gsem[p]).wait()

                def srows(rb, _):
                    for rr in range(8):
                        for c in range(D // 16):
                            sl = pl.ds(c * 16, 16)
                            r = rb * 8 + rr
                            gbuf[p][r, sl] = gbuf[p][r, sl] * SCALE
                    return 0

                lax.fori_loop(0, C // 8, srows, 0)

                row = wid * BPW + ci * C
                pltpu.async_copy(
                    gbuf[p], out_hbm.at[pl.ds(row, C), pl.ds(0, D)], ssem[p])

                q = (k + PF) % NBUF

                @pl.when(ci >= PF)
                def _():
                    store_wait(q)

                @pl.when(ci + PF < NCHUNK)
                def _():
                    gather(ci + PF, q)
            return 0

        lax.fori_loop(0, NCHUNK // NBUF, outer, 0)

        for p in range(NBUF - PF, NBUF):
            store_wait(p)

    return kern(idx3, table)


def kernel(inp, emb_weight):
    idx3 = inp.reshape(NW, NCHUNK, C)
    out128 = _sc_gather(idx3, emb_weight)
    return out128.reshape(NB, NT, 2 * D)[:, :, :D]


# speedup vs baseline: 1.0029x; 1.0029x over previous
import functools

import jax
import jax.numpy as jnp
from jax import lax
from jax.experimental import pallas as pl
from jax.experimental.pallas import tpu as pltpu
from jax.experimental.pallas import tpu_sc as plsc

D = 64
SCALE = 8.0
NB = 4096
NT = 200
B = NB * NT
NC = 2
NS = 16
NW = NC * NS
BPW = B // NW
C = 128
NCHUNK = BPW // C
NBUF = 8
PF = 4


def _sc_gather(idx3, table):
    mesh = plsc.VectorSubcoreMesh(core_axis_name="c", subcore_axis_name="s")

    scratch = [pltpu.VMEM((NCHUNK, C), jnp.int32)]
    scratch += [pltpu.VMEM((C, D), jnp.float32) for _ in range(NBUF)]
    scratch += [pltpu.SemaphoreType.DMA for _ in range(2 * NBUF + 1)]

    @functools.partial(
        pl.kernel,
        mesh=mesh,
        out_type=jax.ShapeDtypeStruct((B, 2 * D), jnp.float32),
        scratch_types=scratch,
        compiler_params=pltpu.CompilerParams(
            use_tc_tiling_on_sc=False, needs_layout_passes=False),
    )
    def kern(idx_hbm, tab_hbm, out_hbm, idx_v, *bufs_and_sems):
        gbuf = bufs_and_sems[:NBUF]
        gsem = bufs_and_sems[NBUF:2 * NBUF]
        ssem = bufs_and_sems[2 * NBUF:3 * NBUF]
        isem = bufs_and_sems[3 * NBUF]

        wid = lax.axis_index("s") * NC + lax.axis_index("c")
        pltpu.async_copy(idx_hbm.at[wid], idx_v, isem).wait()

        def gather(ci, p):
            pltpu.async_copy(tab_hbm.at[idx_v.at[ci]], gbuf[p], gsem[p])

        def store_wait(p):
            pltpu.make_async_copy(
                gbuf[p], out_hbm.at[pl.ds(0, C), pl.ds(0, D)], ssem[p]).wait()

        for p in range(PF):
            gather(p, p)

        def outer(i, _):
            cg = i * NBUF
            for k in range(NBUF):
                ci = cg + k
                p = k
                pltpu.make_async_copy(
                    tab_hbm.at[idx_v.at[ci]], gbuf[p], gsem[p]).wait()

                def srows(rb, _):
                    for rr in range(8):
                        for c in range(D // 16):
                            sl = pl.ds(c * 16, 16)
                            r = rb * 8 + rr
                            gbuf[p][r, sl] = gbuf[p][r, sl] * SCALE
                    return 0

                lax.fori_loop(0, C // 8, srows, 0)

                row = wid * BPW + ci * C
                pltpu.async_copy(
                    gbuf[p], out_hbm.at[pl.ds(row, C), pl.ds(0, D)], ssem[p])

                q = (k + PF) % NBUF

                @pl.when(ci >= PF)
                def _():
                    store_wait(q)

                @pl.when(ci + PF < NCHUNK)
                def _():
                    gather(ci + PF, q)
            return 0

        lax.fori_loop(0, NCHUNK // NBUF, outer, 0)

        for p in range(NBUF - PF, NBUF):
            store_wait(p)

    return kern(idx3, table)


def kernel(inp, emb_weight):
    idx3 = inp.reshape(NW, NCHUNK, C)
    out128 = _sc_gather(idx3, emb_weight)
    return out128.reshape(NB, NT, 2 * D)[:, :, :D]
